# Initial kernel scaffold; baseline (speedup 1.0000x reference)
import functools

import jax
import jax.numpy as jnp
from jax import lax
from jax.experimental import pallas as pl
from jax.experimental.pallas import tpu as pltpu
from jax.experimental.pallas import tpu_sc as plsc

N = 10000
E = 320000
F_IN = 128
H = 16
C = 2
G = 64

NC = 2
NS = 16
NW = NC * NS
EPW = E // NW
CHUNK = 80
NITER = EPW // CHUNK
RPT = N // NS

_mesh = plsc.VectorSubcoreMesh(core_axis_name="c", subcore_axis_name="s")


@functools.partial(
    pl.kernel,
    out_type=jax.ShapeDtypeStruct((NW, N), jnp.float32),
    mesh=_mesh,
    scratch_types=[
        pltpu.VMEM((EPW,), jnp.int32),
        pltpu.VMEM((N,), jnp.float32),
    ],
)
def _sc_degree(dst_hbm, out_hbm, dst_v, deg_v):
    c = lax.axis_index("c")
    s = lax.axis_index("s")
    wid = c * NS + s
    zeros = jnp.zeros((16,), jnp.float32)
    ones = jnp.ones((16,), jnp.float32)

    def zero_body(i, carry):
        deg_v[pl.ds(i * 16, 16)] = zeros
        return carry

    lax.fori_loop(0, N // 16, zero_body, 0)

    pltpu.sync_copy(dst_hbm.at[pl.ds(wid * EPW, EPW)], dst_v)

    def add_body(i, carry):
        iv = dst_v[pl.ds(i * 16, 16)]
        plsc.addupdate_scatter(deg_v, [iv], ones)
        return carry

    lax.fori_loop(0, EPW // 16, add_body, 0)

    pltpu.sync_copy(deg_v, out_hbm.at[wid])


@functools.partial(
    pl.kernel,
    out_type=jax.ShapeDtypeStruct((NC, N, H), jnp.float32),
    mesh=_mesh,
    scratch_types=[
        pltpu.VMEM((CHUNK,), jnp.int32),
        pltpu.VMEM((CHUNK,), jnp.int32),
        pltpu.VMEM((CHUNK, H), jnp.float32),
        pltpu.VMEM((RPT, H), jnp.float32),
        pltpu.VMEM_SHARED((N, H), jnp.float32),
        pltpu.SemaphoreType.DMA,
    ],
)
def _sc_propagate(y_hbm, src_hbm, dst_hbm, out_hbm, src_v, dst_v, msg_v,
                  stage_v, acc_sh, sem):
    c = lax.axis_index("c")
    s = lax.axis_index("s")
    wid = c * NS + s
    zeros = jnp.zeros((16,), jnp.float32)

    def zero_body(i, carry):
        stage_v[i, :] = zeros
        return carry

    lax.fori_loop(0, RPT, zero_body, 0)
    row0 = s * RPT
    pltpu.sync_copy(stage_v, acc_sh.at[pl.ds(row0, RPT)])
    plsc.subcore_barrier()

    base = wid * EPW

    def edge_body(i, carry):
        off = base + i * CHUNK
        pltpu.sync_copy(src_hbm.at[pl.ds(off, CHUNK)], src_v)
        pltpu.sync_copy(dst_hbm.at[pl.ds(off, CHUNK)], dst_v)
        pltpu.async_copy(y_hbm.at[src_v], msg_v, sem).wait()
        pltpu.sync_copy(msg_v, acc_sh.at[dst_v], add=True)
        return carry

    lax.fori_loop(0, NITER, edge_body, 0)

    plsc.subcore_barrier()
    pltpu.sync_copy(acc_sh.at[pl.ds(row0, RPT)],
                    out_hbm.at[c, pl.ds(row0, RPT)])


def _tc_stage1_body(degp_ref, x_ref, w1_ref, dis_ref, y1_ref):
    deg = jnp.sum(degp_ref[...], axis=0) + 1.0
    dis = lax.rsqrt(deg)
    h = jnp.dot(x_ref[...], w1_ref[...], preferred_element_type=jnp.float32)
    y1_ref[...] = h * dis[:, None]
    dis_ref[...] = dis[:, None]


def _tc_stage2_body(y1_ref, p_ref, dis_ref, b1_ref, y2_ref):
    dis = dis_ref[...]
    acc = y1_ref[...] + p_ref[0] + p_ref[1]
    t = acc * dis + b1_ref[...]
    r = jnp.maximum(t, 0.0)
    y2_ref[...] = r * dis


def _tc_stage3_body(y2_ref, p_ref, dis_ref, w2_ref, b2_ref, batch_ref,
                    out_ref):
    dis = dis_ref[...]
    acc = y2_ref[...] + p_ref[0] + p_ref[1]
    z = jnp.dot(acc * dis, w2_ref[...],
                preferred_element_type=jnp.float32) + b2_ref[...]
    b = batch_ref[...]
    m = (b == lax.broadcasted_iota(jnp.int32, (N, G), 1)).astype(jnp.float32)
    sums = lax.dot_general(m, z, (((0,), (0,)), ((), ())),
                           preferred_element_type=jnp.float32)
    counts = jnp.sum(m, axis=0)[:, None]
    pooled = sums / jnp.maximum(counts, 1.0)
    mx = jnp.max(pooled, axis=1, keepdims=True)
    lse = mx + jnp.log(jnp.sum(jnp.exp(pooled - mx), axis=1, keepdims=True))
    out_ref[...] = pooled - lse


_tc_stage1 = pl.pallas_call(
    _tc_stage1_body,
    out_shape=[
        jax.ShapeDtypeStruct((N, 1), jnp.float32),
        jax.ShapeDtypeStruct((N, H), jnp.float32),
    ],
)

_tc_stage2 = pl.pallas_call(
    _tc_stage2_body,
    out_shape=jax.ShapeDtypeStruct((N, H), jnp.float32),
)

_tc_stage3 = pl.pallas_call(
    _tc_stage3_body,
    out_shape=jax.ShapeDtypeStruct((G, C), jnp.float32),
)


@jax.jit
def kernel(x, edge_index, batch, W1, b1, W2, b2):
    src = edge_index[0]
    dst = edge_index[1]
    deg_parts = _sc_degree(dst)
    dis, y1 = _tc_stage1(deg_parts, x, W1)
    p1 = _sc_propagate(y1, src, dst)
    y2 = _tc_stage2(y1, p1, dis, b1.reshape(1, H))
    p2 = _sc_propagate(y2, src, dst)
    return _tc_stage3(y2, p2, dis, W2, b2.reshape(1, C),
                      batch.reshape(N, 1))


# speedup vs baseline: 18.6196x; 18.6196x over previous
import functools

import jax
import jax.numpy as jnp
from jax import lax
from jax.experimental import pallas as pl
from jax.experimental.pallas import tpu as pltpu
from jax.experimental.pallas import tpu_sc as plsc

N = 10000
E = 320000
F_IN = 128
H = 16
C = 2
G = 64

NC = 2
NS = 16
NW = NC * NS
EPW = E // NW
CHUNK = 80
NITER = EPW // CHUNK
NP = 10240
RPT = NP // NS

def _sc_degree_body(dst_hbm, out_hbm, dst_v, deg_v):
    c = lax.axis_index("c")
    s = lax.axis_index("s")
    wid = c * NS + s
    zeros = jnp.zeros((16,), jnp.float32)
    ones = jnp.ones((16,), jnp.float32)

    def zero_body(i, carry):
        deg_v[pl.ds(i * 16, 16)] = zeros
        return carry

    lax.fori_loop(0, N // 16, zero_body, 0)

    pltpu.sync_copy(dst_hbm.at[pl.ds(wid * EPW, EPW)], dst_v)

    def add_body(i, carry):
        iv = dst_v[pl.ds(i * 16, 16)]
        plsc.addupdate_scatter(deg_v, [iv], ones)
        return carry

    lax.fori_loop(0, EPW // 16, add_body, 0)

    pltpu.sync_copy(deg_v, out_hbm.at[pl.ds(wid * N, N)])


def _sc_propagate_body(y_hbm, src_hbm, dst_hbm, out_hbm, src_v, dst_v, msg_v,
                       stage_v, acc_sh, sem):
    c = lax.axis_index("c")
    s = lax.axis_index("s")
    wid = c * NS + s
    zeros = jnp.zeros((16,), jnp.float32)

    def zero_body(i, carry):
        stage_v[i, :] = zeros
        return carry

    lax.fori_loop(0, RPT, zero_body, 0)
    row0 = s * RPT
    pltpu.sync_copy(stage_v, acc_sh.at[pl.ds(row0, RPT)])
    plsc.subcore_barrier()

    base = wid * EPW

    def edge_body(i, carry):
        off = base + i * CHUNK
        pltpu.sync_copy(src_hbm.at[pl.ds(off, CHUNK)], src_v)
        pltpu.sync_copy(dst_hbm.at[pl.ds(off, CHUNK)], dst_v)
        pltpu.async_copy(y_hbm.at[src_v], msg_v, sem).wait()
        pltpu.sync_copy(msg_v, acc_sh.at[dst_v], add=True)
        return carry

    lax.fori_loop(0, NITER, edge_body, 0)

    plsc.subcore_barrier()
    pltpu.sync_copy(acc_sh.at[pl.ds(row0, RPT)],
                    out_hbm.at[c, pl.ds(row0, RPT)])


def _tc_stage1_body(degp_ref, x_ref, w1_ref, dis_ref, y1_ref):
    deg = jnp.sum(degp_ref[...], axis=0) + 1.0
    dis = lax.rsqrt(deg)
    h = jnp.dot(x_ref[...], w1_ref[...], preferred_element_type=jnp.float32)
    y1_ref[...] = h * dis[:, None]
    dis_ref[...] = dis[:, None]


def _tc_stage2_body(y1_ref, p_ref, dis_ref, b1_ref, y2_ref):
    dis = dis_ref[...]
    acc = y1_ref[...] + p_ref[0, :N, :] + p_ref[1, :N, :]
    t = acc * dis + b1_ref[...]
    r = jnp.maximum(t, 0.0)
    y2_ref[...] = r * dis


def _tc_stage3_body(y2_ref, p_ref, dis_ref, w2_ref, b2_ref, batch_ref,
                    out_ref):
    dis = dis_ref[...]
    acc = y2_ref[...] + p_ref[0, :N, :] + p_ref[1, :N, :]
    z = jnp.dot(acc * dis, w2_ref[...],
                preferred_element_type=jnp.float32) + b2_ref[...]
    b = batch_ref[...]
    m = (b == lax.broadcasted_iota(jnp.int32, (N, G), 1)).astype(jnp.float32)
    sums = lax.dot_general(m, z, (((0,), (0,)), ((), ())),
                           preferred_element_type=jnp.float32)
    counts = jnp.sum(m, axis=0)[:, None]
    pooled = sums / jnp.maximum(counts, 1.0)
    mx = jnp.max(pooled, axis=1, keepdims=True)
    lse = mx + jnp.log(jnp.sum(jnp.exp(pooled - mx), axis=1, keepdims=True))
    out_ref[...] = pooled - lse


_tc_stage1 = pl.pallas_call(
    _tc_stage1_body,
    out_shape=[
        jax.ShapeDtypeStruct((N, 1), jnp.float32),
        jax.ShapeDtypeStruct((N, H), jnp.float32),
    ],
)

_tc_stage2 = pl.pallas_call(
    _tc_stage2_body,
    out_shape=jax.ShapeDtypeStruct((N, H), jnp.float32),
)

_tc_stage3 = pl.pallas_call(
    _tc_stage3_body,
    out_shape=jax.ShapeDtypeStruct((G, C), jnp.float32),
)


@functools.cache
def _sc_kernels():
    mesh = plsc.VectorSubcoreMesh(core_axis_name="c", subcore_axis_name="s",
                                  num_cores=NC, num_subcores=NS)
    sc_degree = pl.kernel(
        _sc_degree_body,
        out_type=jax.ShapeDtypeStruct((NW * N,), jnp.float32),
        mesh=mesh,
        scratch_types=[
            pltpu.VMEM((EPW,), jnp.int32),
            pltpu.VMEM((N,), jnp.float32),
        ],
        compiler_params=pltpu.CompilerParams(needs_layout_passes=False),
    )
    sc_propagate = pl.kernel(
        _sc_propagate_body,
        out_type=jax.ShapeDtypeStruct((NC, NP, H), jnp.float32),
        mesh=mesh,
        scratch_types=[
            pltpu.VMEM((CHUNK,), jnp.int32),
            pltpu.VMEM((CHUNK,), jnp.int32),
            pltpu.VMEM((CHUNK, H), jnp.float32),
            pltpu.VMEM((RPT, H), jnp.float32),
            pltpu.VMEM_SHARED((NP, H), jnp.float32),
            pltpu.SemaphoreType.DMA,
        ],
        compiler_params=pltpu.CompilerParams(use_tc_tiling_on_sc=False),
    )
    return sc_degree, sc_propagate


@jax.jit
def kernel(x, edge_index, batch, W1, b1, W2, b2):
    _sc_degree, _sc_propagate = _sc_kernels()
    src = edge_index[0]
    dst = edge_index[1]
    deg_parts = _sc_degree(dst).reshape(NW, N)
    dis, y1 = _tc_stage1(deg_parts, x, W1)
    p1 = _sc_propagate(y1, src, dst)
    y2 = _tc_stage2(y1, p1, dis, b1.reshape(1, H))
    p2 = _sc_propagate(y2, src, dst)
    return _tc_stage3(y2, p2, dis, W2, b2.reshape(1, C),
                      batch.reshape(N, 1))


# speedup vs baseline: 60.8229x; 3.2666x over previous
import functools

import jax
import jax.numpy as jnp
from jax import lax
from jax.experimental import pallas as pl
from jax.experimental.pallas import tpu as pltpu
from jax.experimental.pallas import tpu_sc as plsc

N = 10000
E = 320000
F_IN = 128
H = 16
C = 2
G = 64

NC = 2
NS = 16
NW = NC * NS
EPW = E // NW
CHUNK = 80
NITER = EPW // CHUNK
KB = 5
NBLK = NITER // KB
NPAIR = (NBLK - 1) // 2
NP = 10240
RPT = NP // NS

def _sc_degree_body(dst_hbm, out_hbm, dst_v, deg_v):
    c = lax.axis_index("c")
    s = lax.axis_index("s")
    wid = c * NS + s
    zeros = jnp.zeros((16,), jnp.float32)
    ones = jnp.ones((16,), jnp.float32)

    def zero_body(i, carry):
        deg_v[pl.ds(i * 16, 16)] = zeros
        return carry

    lax.fori_loop(0, N // 16, zero_body, 0)

    pltpu.sync_copy(dst_hbm.at[pl.ds(wid * EPW, EPW)], dst_v)

    def add_body(i, carry):
        iv = dst_v[pl.ds(i * 16, 16)]
        plsc.addupdate_scatter(deg_v, [iv], ones)
        return carry

    lax.fori_loop(0, EPW // 16, add_body, 0)

    pltpu.sync_copy(deg_v, out_hbm.at[pl.ds(wid * N, N)])


def _sc_propagate_body(y_hbm, src_hbm, dst_hbm, out_hbm, src_v, dst2_v, msg_v,
                       stage_v, acc_sh, sem_g0, sem_g1, sem_s0, sem_s1):
    c = lax.axis_index("c")
    s = lax.axis_index("s")
    wid = c * NS + s
    base = wid * EPW
    zeros = jnp.zeros((16,), jnp.float32)

    pltpu.sync_copy(src_hbm.at[pl.ds(base, EPW)], src_v)
    pltpu.sync_copy(dst_hbm.at[pl.ds(wid * NITER, NITER)], dst2_v)

    def zero_body(i, carry):
        stage_v[i, :] = zeros
        return carry

    lax.fori_loop(0, RPT, zero_body, 0)
    row0 = s * RPT
    pltpu.sync_copy(stage_v, acc_sh.at[pl.ds(row0, RPT)])
    plsc.subcore_barrier()

    sem_g = (sem_g0, sem_g1)
    sem_s = (sem_s0, sem_s1)

    def fire_gathers(blk, p):
        for kk in range(KB):
            ch = blk * KB + kk
            idx = src_v.at[pl.ds(ch * CHUNK, CHUNK)]
            pltpu.async_copy(y_hbm.at[idx], msg_v.at[p, kk], sem_g[p])

    def drain_gathers(p):
        for kk in range(KB):
            pltpu.make_async_copy(y_hbm.at[pl.ds(0, CHUNK)],
                                  msg_v.at[p, kk], sem_g[p]).wait()

    def fire_scatters(blk, p):
        for kk in range(KB):
            ch = blk * KB + kk
            pltpu.async_copy(msg_v.at[p, kk], acc_sh.at[dst2_v.at[ch]],
                             sem_s[p], add=True)

    def drain_scatters(p):
        for kk in range(KB):
            pltpu.make_async_copy(y_hbm.at[pl.ds(0, CHUNK)],
                                  msg_v.at[p, kk], sem_s[p]).wait()

    fire_gathers(0, 0)

    def pair_body(i, carry):
        blk0 = i * 2

        @pl.when(i > 0)
        def _():
            drain_scatters(1)

        fire_gathers(blk0 + 1, 1)
        drain_gathers(0)
        fire_scatters(blk0, 0)

        drain_scatters(0)
        fire_gathers(blk0 + 2, 0)
        drain_gathers(1)
        fire_scatters(blk0 + 1, 1)
        return carry

    lax.fori_loop(0, NPAIR, pair_body, 0)

    drain_scatters(1)
    drain_gathers(0)
    fire_scatters(NBLK - 1, 0)
    drain_scatters(0)

    plsc.subcore_barrier()
    pltpu.sync_copy(acc_sh.at[pl.ds(row0, RPT)],
                    out_hbm.at[c, pl.ds(row0, RPT)])


def _tc_stage1_body(degp_ref, x_ref, w1_ref, dis_ref, y1_ref):
    deg = jnp.sum(degp_ref[...], axis=0) + 1.0
    dis = lax.rsqrt(deg)
    h = jnp.dot(x_ref[...], w1_ref[...], preferred_element_type=jnp.float32)
    y1_ref[...] = h * dis[:, None]
    dis_ref[...] = dis[:, None]


def _tc_stage2_body(y1_ref, p_ref, dis_ref, b1_ref, y2_ref):
    dis = dis_ref[...]
    acc = y1_ref[...] + p_ref[0, :N, :] + p_ref[1, :N, :]
    t = acc * dis + b1_ref[...]
    r = jnp.maximum(t, 0.0)
    y2_ref[...] = r * dis


def _tc_stage3_body(y2_ref, p_ref, dis_ref, w2_ref, b2_ref, batch_ref,
                    out_ref):
    dis = dis_ref[...]
    acc = y2_ref[...] + p_ref[0, :N, :] + p_ref[1, :N, :]
    z = jnp.dot(acc * dis, w2_ref[...],
                preferred_element_type=jnp.float32) + b2_ref[...]
    b = batch_ref[...]
    m = (b == lax.broadcasted_iota(jnp.int32, (N, G), 1)).astype(jnp.float32)
    sums = lax.dot_general(m, z, (((0,), (0,)), ((), ())),
                           preferred_element_type=jnp.float32)
    counts = jnp.sum(m, axis=0)[:, None]
    pooled = sums / jnp.maximum(counts, 1.0)
    mx = jnp.max(pooled, axis=1, keepdims=True)
    lse = mx + jnp.log(jnp.sum(jnp.exp(pooled - mx), axis=1, keepdims=True))
    out_ref[...] = pooled - lse


_tc_stage1 = pl.pallas_call(
    _tc_stage1_body,
    out_shape=[
        jax.ShapeDtypeStruct((N, 1), jnp.float32),
        jax.ShapeDtypeStruct((N, H), jnp.float32),
    ],
)

_tc_stage2 = pl.pallas_call(
    _tc_stage2_body,
    out_shape=jax.ShapeDtypeStruct((N, H), jnp.float32),
)

_tc_stage3 = pl.pallas_call(
    _tc_stage3_body,
    out_shape=jax.ShapeDtypeStruct((G, C), jnp.float32),
)


@functools.cache
def _sc_kernels():
    mesh = plsc.VectorSubcoreMesh(core_axis_name="c", subcore_axis_name="s",
                                  num_cores=NC, num_subcores=NS)
    sc_degree = pl.kernel(
        _sc_degree_body,
        out_type=jax.ShapeDtypeStruct((NW * N,), jnp.float32),
        mesh=mesh,
        scratch_types=[
            pltpu.VMEM((EPW,), jnp.int32),
            pltpu.VMEM((N,), jnp.float32),
        ],
        compiler_params=pltpu.CompilerParams(needs_layout_passes=False),
    )
    sc_propagate = pl.kernel(
        _sc_propagate_body,
        out_type=jax.ShapeDtypeStruct((NC, NP, H), jnp.float32),
        mesh=mesh,
        scratch_types=[
            pltpu.VMEM((EPW,), jnp.int32),
            pltpu.VMEM((NITER, CHUNK), jnp.int32),
            pltpu.VMEM((2, KB, CHUNK, H), jnp.float32),
            pltpu.VMEM((RPT, H), jnp.float32),
            pltpu.VMEM_SHARED((NP, H), jnp.float32),
            pltpu.SemaphoreType.DMA,
            pltpu.SemaphoreType.DMA,
            pltpu.SemaphoreType.DMA,
            pltpu.SemaphoreType.DMA,
        ],
        compiler_params=pltpu.CompilerParams(use_tc_tiling_on_sc=False),
    )
    return sc_degree, sc_propagate


@jax.jit
def kernel(x, edge_index, batch, W1, b1, W2, b2):
    _sc_degree, _sc_propagate = _sc_kernels()
    src = edge_index[0]
    dst = edge_index[1]
    deg_parts = _sc_degree(dst).reshape(NW, N)
    dis, y1 = _tc_stage1(deg_parts, x, W1)
    dst2 = dst.reshape(E // CHUNK, CHUNK)
    p1 = _sc_propagate(y1, src, dst2)
    y2 = _tc_stage2(y1, p1, dis, b1.reshape(1, H))
    p2 = _sc_propagate(y2, src, dst2)
    return _tc_stage3(y2, p2, dis, W2, b2.reshape(1, C),
                      batch.reshape(N, 1))


# speedup vs baseline: 65.4318x; 1.0758x over previous
import functools

import jax
import jax.numpy as jnp
from jax import lax
from jax.experimental import pallas as pl
from jax.experimental.pallas import tpu as pltpu
from jax.experimental.pallas import tpu_sc as plsc

N = 10000
E = 320000
F_IN = 128
H = 16
C = 2
G = 64

NC = 2
NS = 16
NW = NC * NS
EPW = E // NW
CHUNK = 80
NITER = EPW // CHUNK
KB = 5
NBLK = NITER // KB
NPAIR = (NBLK - 1) // 2
NP = 10240
RPT = NP // NS

def _sc_degree_body(ei_hbm, out_hbm, dst_v, deg_v):
    c = lax.axis_index("c")
    s = lax.axis_index("s")
    wid = c * NS + s
    zeros = jnp.zeros((16,), jnp.float32)
    ones = jnp.ones((16,), jnp.float32)

    def zero_body(i, carry):
        deg_v[pl.ds(i * 16, 16)] = zeros
        return carry

    lax.fori_loop(0, N // 16, zero_body, 0)

    pltpu.sync_copy(ei_hbm.at[1, pl.ds(wid * NITER, NITER)], dst_v)

    def add_body(i, carry):
        for k in range(CHUNK // 16):
            iv = dst_v[i, pl.ds(k * 16, 16)]
            plsc.addupdate_scatter(deg_v, [iv], ones)
        return carry

    lax.fori_loop(0, NITER, add_body, 0)

    pltpu.sync_copy(deg_v, out_hbm.at[pl.ds(wid * N, N)])


def _sc_propagate_body(y_hbm, ei_hbm, out_hbm, src2_v, dst2_v, msg_v,
                       stage_v, acc_sh, sem_g0, sem_g1, sem_s0, sem_s1):
    c = lax.axis_index("c")
    s = lax.axis_index("s")
    wid = c * NS + s
    zeros = jnp.zeros((16,), jnp.float32)

    pltpu.sync_copy(ei_hbm.at[0, pl.ds(wid * NITER, NITER)], src2_v)
    pltpu.sync_copy(ei_hbm.at[1, pl.ds(wid * NITER, NITER)], dst2_v)

    def zero_body(i, carry):
        stage_v[i, :] = zeros
        return carry

    lax.fori_loop(0, RPT, zero_body, 0)
    row0 = s * RPT
    pltpu.sync_copy(stage_v, acc_sh.at[pl.ds(row0, RPT)])
    plsc.subcore_barrier()

    sem_g = (sem_g0, sem_g1)
    sem_s = (sem_s0, sem_s1)

    def fire_gathers(blk, p):
        for kk in range(KB):
            ch = blk * KB + kk
            pltpu.async_copy(y_hbm.at[src2_v.at[ch]], msg_v.at[p, kk],
                             sem_g[p])

    def drain_gathers(p):
        for kk in range(KB):
            pltpu.make_async_copy(y_hbm.at[pl.ds(0, CHUNK)],
                                  msg_v.at[p, kk], sem_g[p]).wait()

    def fire_scatters(blk, p):
        for kk in range(KB):
            ch = blk * KB + kk
            pltpu.async_copy(msg_v.at[p, kk], acc_sh.at[dst2_v.at[ch]],
                             sem_s[p], add=True)

    def drain_scatters(p):
        for kk in range(KB):
            pltpu.make_async_copy(y_hbm.at[pl.ds(0, CHUNK)],
                                  msg_v.at[p, kk], sem_s[p]).wait()

    fire_gathers(0, 0)

    def pair_body(i, carry):
        blk0 = i * 2

        @pl.when(i > 0)
        def _():
            drain_scatters(1)

        fire_gathers(blk0 + 1, 1)
        drain_gathers(0)
        fire_scatters(blk0, 0)

        drain_scatters(0)
        fire_gathers(blk0 + 2, 0)
        drain_gathers(1)
        fire_scatters(blk0 + 1, 1)
        return carry

    lax.fori_loop(0, NPAIR, pair_body, 0)

    drain_scatters(1)
    drain_gathers(0)
    fire_scatters(NBLK - 1, 0)
    drain_scatters(0)

    plsc.subcore_barrier()
    pltpu.sync_copy(acc_sh.at[pl.ds(row0, RPT)],
                    out_hbm.at[c, pl.ds(row0, RPT)])


def _tc_stage1_body(degp_ref, x_ref, w1_ref, dis_ref, y1_ref):
    deg = jnp.sum(degp_ref[...], axis=0) + 1.0
    dis = lax.rsqrt(deg)
    h = jnp.dot(x_ref[...], w1_ref[...], preferred_element_type=jnp.float32)
    y1_ref[...] = h * dis[:, None]
    dis_ref[...] = dis[:, None]


def _tc_stage2_body(y1_ref, p_ref, dis_ref, b1_ref, y2_ref):
    dis = dis_ref[...]
    acc = y1_ref[...] + p_ref[0, :N, :] + p_ref[1, :N, :]
    t = acc * dis + b1_ref[...]
    r = jnp.maximum(t, 0.0)
    y2_ref[...] = r * dis


def _tc_stage3_body(y2_ref, p_ref, dis_ref, w2_ref, b2_ref, batch_ref,
                    out_ref):
    dis = dis_ref[...]
    acc = y2_ref[...] + p_ref[0, :N, :] + p_ref[1, :N, :]
    z = jnp.dot(acc * dis, w2_ref[...],
                preferred_element_type=jnp.float32) + b2_ref[...]
    b = batch_ref[...]
    m = (b == lax.broadcasted_iota(jnp.int32, (N, G), 1)).astype(jnp.float32)
    sums = lax.dot_general(m, z, (((0,), (0,)), ((), ())),
                           preferred_element_type=jnp.float32)
    counts = jnp.sum(m, axis=0)[:, None]
    pooled = sums / jnp.maximum(counts, 1.0)
    mx = jnp.max(pooled, axis=1, keepdims=True)
    lse = mx + jnp.log(jnp.sum(jnp.exp(pooled - mx), axis=1, keepdims=True))
    out_ref[...] = pooled - lse


_tc_stage1 = pl.pallas_call(
    _tc_stage1_body,
    out_shape=[
        jax.ShapeDtypeStruct((N, 1), jnp.float32),
        jax.ShapeDtypeStruct((N, H), jnp.float32),
    ],
)

_tc_stage2 = pl.pallas_call(
    _tc_stage2_body,
    out_shape=jax.ShapeDtypeStruct((N, H), jnp.float32),
)

_tc_stage3 = pl.pallas_call(
    _tc_stage3_body,
    out_shape=jax.ShapeDtypeStruct((G, C), jnp.float32),
)


@functools.cache
def _sc_kernels():
    mesh = plsc.VectorSubcoreMesh(core_axis_name="c", subcore_axis_name="s",
                                  num_cores=NC, num_subcores=NS)
    sc_degree = pl.kernel(
        _sc_degree_body,
        out_type=jax.ShapeDtypeStruct((NW * N,), jnp.float32),
        mesh=mesh,
        scratch_types=[
            pltpu.VMEM((NITER, CHUNK), jnp.int32),
            pltpu.VMEM((N,), jnp.float32),
        ],
        compiler_params=pltpu.CompilerParams(needs_layout_passes=False,
                                             use_tc_tiling_on_sc=False),
    )
    sc_propagate = pl.kernel(
        _sc_propagate_body,
        out_type=jax.ShapeDtypeStruct((NC, NP, H), jnp.float32),
        mesh=mesh,
        scratch_types=[
            pltpu.VMEM((NITER, CHUNK), jnp.int32),
            pltpu.VMEM((NITER, CHUNK), jnp.int32),
            pltpu.VMEM((2, KB, CHUNK, H), jnp.float32),
            pltpu.VMEM((RPT, H), jnp.float32),
            pltpu.VMEM_SHARED((NP, H), jnp.float32),
            pltpu.SemaphoreType.DMA,
            pltpu.SemaphoreType.DMA,
            pltpu.SemaphoreType.DMA,
            pltpu.SemaphoreType.DMA,
        ],
        compiler_params=pltpu.CompilerParams(use_tc_tiling_on_sc=False),
    )
    return sc_degree, sc_propagate


@jax.jit
def kernel(x, edge_index, batch, W1, b1, W2, b2):
    _sc_degree, _sc_propagate = _sc_kernels()
    ei3 = edge_index.reshape(2, E // CHUNK, CHUNK)
    deg_parts = _sc_degree(ei3).reshape(NW, N)
    dis, y1 = _tc_stage1(deg_parts, x, W1)
    p1 = _sc_propagate(y1, ei3)
    y2 = _tc_stage2(y1, p1, dis, b1.reshape(1, H))
    p2 = _sc_propagate(y2, ei3)
    return _tc_stage3(y2, p2, dis, W2, b2.reshape(1, C),
                      batch.reshape(N, 1))


# speedup vs baseline: 81.0271x; 1.2383x over previous
import functools

import jax
import jax.numpy as jnp
from jax import lax
from jax.experimental import pallas as pl
from jax.experimental.pallas import tpu as pltpu
from jax.experimental.pallas import tpu_sc as plsc

N = 10000
E = 320000
F_IN = 128
H = 16
C = 2
G = 64

NC = 2
NS = 16
COLS = H // NC
EPT = E // NS
CHUNK = 80
NIT2 = EPT // CHUNK
KB = 10
NBLK = NIT2 // KB
NPAIR = (NBLK - 1) // 2
NP = 10240
RPT = NP // NS


def _sc_fused_body(h_hbm, ei_hbm, b1_hbm, out_hbm, src2_v, dst2_v, msg_v,
                   hist_v, hred_v, dis_v, hs_v, ys_v, b1_v,
                   hist_sh, table_sh, acc_sh, sem_g0, sem_g1, sem_s0,
                   sem_s1):
    c = lax.axis_index("c")
    s = lax.axis_index("s")
    row0 = s * RPT
    zeros16 = jnp.zeros((16,), jnp.float32)
    ones16 = jnp.ones((16,), jnp.float32)
    lane = lax.iota(jnp.int32, 16)
    lane_pair = lax.shift_right_logical(lane, 3)
    lane_col = lane & 7

    pltpu.sync_copy(ei_hbm.at[0, pl.ds(s * NIT2, NIT2)], src2_v)
    pltpu.sync_copy(ei_hbm.at[1, pl.ds(s * NIT2, NIT2)], dst2_v)
    pltpu.sync_copy(b1_hbm.at[c], b1_v)

    def zero_hist(i, carry):
        hist_v[pl.ds(i * 16, 16)] = zeros16
        return carry

    lax.fori_loop(0, NP // 16, zero_hist, 0)

    def deg_body(i, carry):
        for k in range(CHUNK // 16):
            iv = dst2_v[i, pl.ds(k * 16, 16)]
            plsc.addupdate_scatter(hist_v, [iv], ones16)
        return carry

    lax.fori_loop(0, NIT2, deg_body, 0)

    pltpu.sync_copy(hist_v, hist_sh.at[s])
    plsc.subcore_barrier()

    for t in range(NS):
        pltpu.sync_copy(hist_sh.at[t, pl.ds(row0, RPT)], hred_v.at[t])

    magic = jnp.full((16,), 0x5F3759DF, jnp.int32)

    def dis_body(j, carry):
        d = hred_v[0, pl.ds(j * 16, 16)]
        for t in range(1, NS):
            d = d + hred_v[t, pl.ds(j * 16, 16)]
        d = d + 1.0
        bits = plsc.bitcast(d, jnp.int32)
        y = plsc.bitcast(magic - lax.shift_right_logical(bits, 1),
                         jnp.float32)
        for _ in range(3):
            y = y * (1.5 - 0.5 * d * y * y)
        dis_v[pl.ds(j * 16, 16)] = y
        return carry

    lax.fori_loop(0, RPT // 16, dis_body, 0)

    pltpu.sync_copy(h_hbm.at[c, pl.ds(row0, RPT)], hs_v)

    def l1_body(j, carry):
        idxr = lane_pair + j * 2
        dv = plsc.load_gather(dis_v, [idxr])
        hv = plsc.load_gather(hs_v, [idxr, lane_col])
        plsc.store_scatter(ys_v, [idxr, lane_col], hv * dv)
        return carry

    lax.fori_loop(0, RPT // 2, l1_body, 0)
    pltpu.sync_copy(ys_v, table_sh.at[pl.ds(row0, RPT)])
    pltpu.sync_copy(ys_v, acc_sh.at[pl.ds(row0, RPT)])
    plsc.subcore_barrier()

    sem_g = (sem_g0, sem_g1)
    sem_s = (sem_s0, sem_s1)

    def fire_gathers(blk, p):
        for kk in range(KB):
            ch = blk * KB + kk
            pltpu.async_copy(table_sh.at[src2_v.at[ch]], msg_v.at[p, kk],
                             sem_g[p])

    def drain_gathers(p):
        for kk in range(KB):
            pltpu.make_async_copy(table_sh.at[pl.ds(0, CHUNK)],
                                  msg_v.at[p, kk], sem_g[p]).wait()

    def fire_scatters(blk, p):
        for kk in range(KB):
            ch = blk * KB + kk
            pltpu.async_copy(msg_v.at[p, kk], acc_sh.at[dst2_v.at[ch]],
                             sem_s[p], add=True)

    def drain_scatters(p):
        for kk in range(KB):
            pltpu.make_async_copy(table_sh.at[pl.ds(0, CHUNK)],
                                  msg_v.at[p, kk], sem_s[p]).wait()

    def edge_loop():
        fire_gathers(0, 0)

        def pair_body(i, carry):
            blk0 = i * 2

            @pl.when(i > 0)
            def _():
                drain_scatters(1)

            fire_gathers(blk0 + 1, 1)
            drain_gathers(0)
            fire_scatters(blk0, 0)

            drain_scatters(0)
            fire_gathers(blk0 + 2, 0)
            drain_gathers(1)
            fire_scatters(blk0 + 1, 1)
            return carry

        lax.fori_loop(0, NPAIR, pair_body, 0)

        drain_scatters(1)
        drain_gathers(0)
        fire_scatters(NBLK - 1, 0)
        drain_scatters(0)

    edge_loop()
    plsc.subcore_barrier()

    pltpu.sync_copy(acc_sh.at[pl.ds(row0, RPT)], hs_v)
    b1v = b1_v[...]

    def relu_body(j, carry):
        idxr = lane_pair + j * 2
        dv = plsc.load_gather(dis_v, [idxr])
        av = plsc.load_gather(hs_v, [idxr, lane_col])
        r = jnp.maximum(av * dv + b1v, 0.0)
        plsc.store_scatter(ys_v, [idxr, lane_col], r * dv)
        return carry

    lax.fori_loop(0, RPT // 2, relu_body, 0)
    pltpu.sync_copy(ys_v, table_sh.at[pl.ds(row0, RPT)])
    pltpu.sync_copy(ys_v, acc_sh.at[pl.ds(row0, RPT)])
    plsc.subcore_barrier()

    edge_loop()
    plsc.subcore_barrier()

    pltpu.sync_copy(acc_sh.at[pl.ds(row0, RPT)], hs_v)

    def out_body(j, carry):
        idxr = lane_pair + j * 2
        dv = plsc.load_gather(dis_v, [idxr])
        av = plsc.load_gather(hs_v, [idxr, lane_col])
        plsc.store_scatter(ys_v, [idxr, lane_col], av * dv)
        return carry

    lax.fori_loop(0, RPT // 2, out_body, 0)
    pltpu.sync_copy(ys_v, out_hbm.at[c, pl.ds(row0, RPT)])


def _tc_a_body(x_ref, w1_ref, h_ref):
    h = jnp.dot(x_ref[...], w1_ref[...], preferred_element_type=jnp.float32)
    h_ref[0, :N, :] = h[:, 0:COLS]
    h_ref[1, :N, :] = h[:, COLS:H]
    h_ref[0, N:NP, :] = jnp.zeros((NP - N, COLS), jnp.float32)
    h_ref[1, N:NP, :] = jnp.zeros((NP - N, COLS), jnp.float32)


def _tc_b_body(p_ref, w2_ref, b2_ref, batch_ref, out_ref):
    z16 = jnp.concatenate([p_ref[0, :N, :], p_ref[1, :N, :]], axis=1)
    z = jnp.dot(z16, w2_ref[...],
                preferred_element_type=jnp.float32) + b2_ref[...]
    b = batch_ref[...]
    m = (b == lax.broadcasted_iota(jnp.int32, (N, G), 1)).astype(jnp.float32)
    sums = lax.dot_general(m, z, (((0,), (0,)), ((), ())),
                           preferred_element_type=jnp.float32)
    counts = jnp.sum(m, axis=0)[:, None]
    pooled = sums / jnp.maximum(counts, 1.0)
    mx = jnp.max(pooled, axis=1, keepdims=True)
    lse = mx + jnp.log(jnp.sum(jnp.exp(pooled - mx), axis=1, keepdims=True))
    out_ref[...] = pooled - lse


_tc_a = pl.pallas_call(
    _tc_a_body,
    out_shape=jax.ShapeDtypeStruct((NC, NP, COLS), jnp.float32),
)

_tc_b = pl.pallas_call(
    _tc_b_body,
    out_shape=jax.ShapeDtypeStruct((G, C), jnp.float32),
)


@functools.cache
def _sc_kernels():
    mesh = plsc.VectorSubcoreMesh(core_axis_name="c", subcore_axis_name="s",
                                  num_cores=NC, num_subcores=NS)
    sc_fused = pl.kernel(
        _sc_fused_body,
        out_type=jax.ShapeDtypeStruct((NC, NP, COLS), jnp.float32),
        mesh=mesh,
        scratch_types=[
            pltpu.VMEM((NIT2, CHUNK), jnp.int32),
            pltpu.VMEM((NIT2, CHUNK), jnp.int32),
            pltpu.VMEM((2, KB, CHUNK, COLS), jnp.float32),
            pltpu.VMEM((NP,), jnp.float32),
            pltpu.VMEM((NS, RPT), jnp.float32),
            pltpu.VMEM((RPT,), jnp.float32),
            pltpu.VMEM((RPT, COLS), jnp.float32),
            pltpu.VMEM((RPT, COLS), jnp.float32),
            pltpu.VMEM((16,), jnp.float32),
            pltpu.VMEM_SHARED((NS, NP), jnp.float32),
            pltpu.VMEM_SHARED((NP, COLS), jnp.float32),
            pltpu.VMEM_SHARED((NP, COLS), jnp.float32),
            pltpu.SemaphoreType.DMA,
            pltpu.SemaphoreType.DMA,
            pltpu.SemaphoreType.DMA,
            pltpu.SemaphoreType.DMA,
        ],
        compiler_params=pltpu.CompilerParams(needs_layout_passes=False,
                                             use_tc_tiling_on_sc=False),
    )
    return sc_fused


@jax.jit
def kernel(x, edge_index, batch, W1, b1, W2, b2):
    sc_fused = _sc_kernels()
    ei3 = edge_index.reshape(2, E // CHUNK, CHUNK)
    h = _tc_a(x, W1)
    b1pat = jnp.stack([jnp.tile(b1[0:COLS], 2), jnp.tile(b1[COLS:H], 2)])
    out8 = sc_fused(h, ei3, b1pat)
    return _tc_b(out8, W2, b2.reshape(1, C), batch.reshape(N, 1))


# speedup vs baseline: 88.2729x; 1.0894x over previous
import functools

import jax
import jax.numpy as jnp
from jax import lax
from jax.experimental import pallas as pl
from jax.experimental.pallas import tpu as pltpu
from jax.experimental.pallas import tpu_sc as plsc

N = 10000
E = 320000
F_IN = 128
H = 16
C = 2
G = 64

NC = 2
NS = 16
COLS = H // NC
CHUNK = 128
NROWS = E // CHUNK
NIT2 = NROWS // NS
NEXTRA = NROWS - NIT2 * NS
KB = 12
NBLK = NIT2 // KB
NPAIR = (NBLK - 1) // 2
NP = 10240
RPT = NP // NS
SROWS = RPT * COLS // 128


def _sc_fused_body(h_hbm, ei_hbm, b1_hbm, batch_hbm, out_hbm, src2_v, dst2_v,
                   msg_v, hist_v, hred_v, dis_v, hs_v, ys_v, batch_v, pool_v,
                   b1_v, hist_sh, table_sh, acc_sh, sem_g0, sem_g1, sem_s0,
                   sem_s1):
    c = lax.axis_index("c")
    s = lax.axis_index("s")
    row0 = s * RPT
    zeros16 = jnp.zeros((16,), jnp.float32)
    ones16 = jnp.ones((16,), jnp.float32)
    lane = lax.iota(jnp.int32, 16)
    lane_pair = lax.shift_right_logical(lane, 3)
    lane_col = lane & 7
    has_extra = s < NEXTRA

    pltpu.sync_copy(ei_hbm.at[0, pl.ds(s * NIT2, NIT2)],
                    src2_v.at[pl.ds(0, NIT2)])
    pltpu.sync_copy(ei_hbm.at[1, pl.ds(s * NIT2, NIT2)],
                    dst2_v.at[pl.ds(0, NIT2)])

    @pl.when(has_extra)
    def _():
        pltpu.sync_copy(ei_hbm.at[0, pl.ds(NIT2 * NS + s, 1)],
                        src2_v.at[pl.ds(NIT2, 1)])
        pltpu.sync_copy(ei_hbm.at[1, pl.ds(NIT2 * NS + s, 1)],
                        dst2_v.at[pl.ds(NIT2, 1)])

    pltpu.sync_copy(b1_hbm.at[c], b1_v)

    @pl.when(s < NS - 1)
    def _():
        pltpu.sync_copy(batch_hbm.at[pl.ds(row0, RPT)], batch_v)

    @pl.when(s == NS - 1)
    def _():
        nlast = N - (NS - 1) * RPT
        pltpu.sync_copy(batch_hbm.at[pl.ds(row0, nlast)],
                        batch_v.at[pl.ds(0, nlast)])

    def zero_hist(i, carry):
        hist_v[pl.ds(i * 16, 16)] = zeros16
        return carry

    lax.fori_loop(0, NP // 16, zero_hist, 0)

    def deg_body(i, carry):
        for k in range(CHUNK // 16):
            iv = dst2_v[i, pl.ds(k * 16, 16)]
            plsc.addupdate_scatter(hist_v, [iv], ones16)
        return carry

    lax.fori_loop(0, NIT2, deg_body, 0)

    @pl.when(has_extra)
    def _():
        for k in range(CHUNK // 16):
            iv = dst2_v[NIT2, pl.ds(k * 16, 16)]
            plsc.addupdate_scatter(hist_v, [iv], ones16)

    pltpu.sync_copy(hist_v, hist_sh.at[s])
    plsc.subcore_barrier()

    for t in range(NS):
        pltpu.sync_copy(hist_sh.at[t, pl.ds(row0, RPT)],
                        hred_v.at[t, pl.ds(0, RPT)])

    magic = jnp.full((16,), 0x5F3759DF, jnp.int32)

    def dis_body(j, carry):
        d = hred_v[0, pl.ds(j * 16, 16)]
        for t in range(1, NS):
            d = d + hred_v[t, pl.ds(j * 16, 16)]
        d = d + 1.0
        bits = plsc.bitcast(d, jnp.int32)
        y = plsc.bitcast(magic - lax.shift_right_logical(bits, 1),
                         jnp.float32)
        for _ in range(3):
            y = y * (1.5 - 0.5 * d * y * y)
        dis_v[pl.ds(j * 16, 16)] = y
        return carry

    lax.fori_loop(0, RPT // 16, dis_body, 0)

    pltpu.sync_copy(h_hbm.at[c, pl.ds(row0, RPT)], hs_v)

    def l1_body(j, carry):
        idxr = lane_pair + j * 2
        dv = plsc.load_gather(dis_v, [idxr])
        hv = plsc.load_gather(hs_v, [idxr, lane_col])
        plsc.store_scatter(ys_v, [idxr, lane_col], hv * dv)
        return carry

    lax.fori_loop(0, RPT // 2, l1_body, 0)
    pltpu.sync_copy(ys_v, table_sh.at[pl.ds(row0, RPT)])
    pltpu.sync_copy(ys_v, acc_sh.at[pl.ds(row0, RPT)])
    plsc.subcore_barrier()

    sem_g = (sem_g0, sem_g1)
    sem_s = (sem_s0, sem_s1)

    def fire_gathers(blk, p):
        for kk in range(KB):
            ch = blk * KB + kk
            pltpu.async_copy(table_sh.at[src2_v.at[ch]], msg_v.at[p, kk],
                             sem_g[p])

    def drain_gathers(p):
        for kk in range(KB):
            pltpu.make_async_copy(table_sh.at[pl.ds(0, CHUNK)],
                                  msg_v.at[p, kk], sem_g[p]).wait()

    def fire_scatters(blk, p):
        for kk in range(KB):
            ch = blk * KB + kk
            pltpu.async_copy(msg_v.at[p, kk], acc_sh.at[dst2_v.at[ch]],
                             sem_s[p], add=True)

    def drain_scatters(p):
        for kk in range(KB):
            pltpu.make_async_copy(table_sh.at[pl.ds(0, CHUNK)],
                                  msg_v.at[p, kk], sem_s[p]).wait()

    def edge_loop():
        fire_gathers(0, 0)

        def pair_body(i, carry):
            blk0 = i * 2

            @pl.when(i > 0)
            def _():
                drain_scatters(1)

            fire_gathers(blk0 + 1, 1)
            drain_gathers(0)
            fire_scatters(blk0, 0)

            drain_scatters(0)
            fire_gathers(blk0 + 2, 0)
            drain_gathers(1)
            fire_scatters(blk0 + 1, 1)
            return carry

        lax.fori_loop(0, NPAIR, pair_body, 0)

        drain_scatters(1)
        drain_gathers(0)
        fire_scatters(NBLK - 1, 0)
        drain_scatters(0)

        @pl.when(has_extra)
        def _():
            pltpu.async_copy(table_sh.at[src2_v.at[NIT2]], msg_v.at[0, 0],
                             sem_g0).wait()
            pltpu.async_copy(msg_v.at[0, 0], acc_sh.at[dst2_v.at[NIT2]],
                             sem_s0, add=True).wait()

    edge_loop()
    plsc.subcore_barrier()

    pltpu.sync_copy(acc_sh.at[pl.ds(row0, RPT)], hs_v)
    b1v = b1_v[...]

    def relu_body(j, carry):
        idxr = lane_pair + j * 2
        dv = plsc.load_gather(dis_v, [idxr])
        av = plsc.load_gather(hs_v, [idxr, lane_col])
        r = jnp.maximum(av * dv + b1v, 0.0)
        plsc.store_scatter(ys_v, [idxr, lane_col], r * dv)
        return carry

    lax.fori_loop(0, RPT // 2, relu_body, 0)
    pltpu.sync_copy(ys_v, table_sh.at[pl.ds(row0, RPT)])
    pltpu.sync_copy(ys_v, acc_sh.at[pl.ds(row0, RPT)])
    plsc.subcore_barrier()

    edge_loop()
    plsc.subcore_barrier()

    def zero_pool(i, carry):
        pool_v[pl.ds(i * 16, 16)] = zeros16
        return carry

    lax.fori_loop(0, 2 * G * COLS // 16, zero_pool, 0)

    pltpu.sync_copy(acc_sh.at[pl.ds(row0, RPT)], hs_v)

    def pool_body(j, carry):
        idxr = lane_pair + j * 2
        dv = plsc.load_gather(dis_v, [idxr])
        av = plsc.load_gather(hs_v, [idxr, lane_col])
        seg = plsc.load_gather(batch_v, [idxr])
        idx = seg * COLS + lane_col
        plsc.addupdate_scatter(pool_v, [idx], av * dv)
        plsc.addupdate_scatter(pool_v, [idx + G * COLS], ones16)
        return carry

    npairs = jnp.where(s == NS - 1, (N - (NS - 1) * RPT) // 2, RPT // 2)
    lax.fori_loop(0, npairs, pool_body, 0)

    pltpu.sync_copy(pool_v, hist_sh.at[s, pl.ds(0, 2 * G * COLS)])
    plsc.subcore_barrier()

    @pl.when(s == 0)
    def _():
        for t in range(NS):
            pltpu.sync_copy(hist_sh.at[t, pl.ds(0, 2 * G * COLS)],
                            hred_v.at[t, pl.ds(0, 2 * G * COLS)])

        def red_body(j, carry):
            v = hred_v[0, pl.ds(j * 16, 16)]
            for t in range(1, NS):
                v = v + hred_v[t, pl.ds(j * 16, 16)]
            pool_v[pl.ds(j * 16, 16)] = v
            return carry

        lax.fori_loop(0, 2 * G * COLS // 16, red_body, 0)
        pltpu.sync_copy(pool_v, out_hbm.at[c])


def _tc_a_body(x_ref, w1_ref, h_ref):
    h = jnp.dot(x_ref[...], w1_ref[...], preferred_element_type=jnp.float32)
    h_ref[0, :N, :] = h[:, 0:COLS]
    h_ref[1, :N, :] = h[:, COLS:H]
    h_ref[0, N:NP, :] = jnp.zeros((NP - N, COLS), jnp.float32)
    h_ref[1, N:NP, :] = jnp.zeros((NP - N, COLS), jnp.float32)


def _tc_b_body(sums_ref, counts_ref, w2_ref, b2_ref, out_ref):
    sums16 = jnp.concatenate([sums_ref[0], sums_ref[1]], axis=1)
    z = jnp.dot(sums16, w2_ref[...], preferred_element_type=jnp.float32)
    pooled = z / jnp.maximum(counts_ref[...], 1.0) + b2_ref[...]
    mx = jnp.max(pooled, axis=1, keepdims=True)
    lse = mx + jnp.log(jnp.sum(jnp.exp(pooled - mx), axis=1, keepdims=True))
    out_ref[...] = pooled - lse


_tc_a = pl.pallas_call(
    _tc_a_body,
    out_shape=jax.ShapeDtypeStruct((NC, NP, COLS), jnp.float32),
)

_tc_b = pl.pallas_call(
    _tc_b_body,
    out_shape=jax.ShapeDtypeStruct((G, C), jnp.float32),
)


@functools.cache
def _sc_kernels():
    mesh = plsc.VectorSubcoreMesh(core_axis_name="c", subcore_axis_name="s",
                                  num_cores=NC, num_subcores=NS)
    sc_fused = pl.kernel(
        _sc_fused_body,
        out_type=jax.ShapeDtypeStruct((NC, 2 * G * COLS), jnp.float32),
        mesh=mesh,
        scratch_types=[
            pltpu.VMEM((NIT2 + 1, CHUNK), jnp.int32),
            pltpu.VMEM((NIT2 + 1, CHUNK), jnp.int32),
            pltpu.VMEM((2, KB, CHUNK, COLS), jnp.float32),
            pltpu.VMEM((NP,), jnp.float32),
            pltpu.VMEM((NS, 2 * G * COLS), jnp.float32),
            pltpu.VMEM((RPT,), jnp.float32),
            pltpu.VMEM((RPT, COLS), jnp.float32),
            pltpu.VMEM((RPT, COLS), jnp.float32),
            pltpu.VMEM((RPT,), jnp.int32),
            pltpu.VMEM((2 * G * COLS,), jnp.float32),
            pltpu.VMEM((16,), jnp.float32),
            pltpu.VMEM_SHARED((NS, NP), jnp.float32),
            pltpu.VMEM_SHARED((NP, COLS), jnp.float32),
            pltpu.VMEM_SHARED((NP, COLS), jnp.float32),
            pltpu.SemaphoreType.DMA,
            pltpu.SemaphoreType.DMA,
            pltpu.SemaphoreType.DMA,
            pltpu.SemaphoreType.DMA,
        ],
        compiler_params=pltpu.CompilerParams(needs_layout_passes=False,
                                             use_tc_tiling_on_sc=False),
    )
    return sc_fused


@jax.jit
def kernel(x, edge_index, batch, W1, b1, W2, b2):
    sc_fused = _sc_kernels()
    ei3 = edge_index.reshape(2, NROWS, CHUNK)
    h = _tc_a(x, W1)
    b1pat = jnp.stack([jnp.tile(b1[0:COLS], 2), jnp.tile(b1[COLS:H], 2)])
    pooled = sc_fused(h, ei3, b1pat, batch)
    sums = pooled[:, :G * COLS].reshape(NC, G, COLS)
    counts = pooled[0, G * COLS:].reshape(G, COLS)[:, 0:1]
    return _tc_b(sums, counts, W2, b2.reshape(1, C))


# speedup vs baseline: 97.9380x; 1.1095x over previous
import functools

import jax
import jax.numpy as jnp
from jax import lax
from jax.experimental import pallas as pl
from jax.experimental.pallas import tpu as pltpu
from jax.experimental.pallas import tpu_sc as plsc

N = 10000
E = 320000
F_IN = 128
H = 16
C = 2
G = 64

NC = 2
NS = 16
COLS = H // NC
CHUNK = 128
NROWS = E // CHUNK
NIT2 = NROWS // NS
NEXTRA = NROWS - NIT2 * NS
KB = 12
NBLK = NIT2 // KB
NPAIR = (NBLK - 1) // 2
NP = 10240
RPT = NP // NS
SROWS = RPT * COLS // 128


def _sc_fused_body(h_hbm, ei_hbm, b1_hbm, batch_hbm, out_hbm, src2_v, dst2_v,
                   msg_v, hist_v, hred_v, dis_v, hs_v, ys_v, hsT_v, batch_v,
                   pool_v, b1_v, hist_sh, table_sh, acc_sh, sem_g0, sem_g1,
                   sem_s0, sem_s1):
    c = lax.axis_index("c")
    s = lax.axis_index("s")
    row0 = s * RPT
    zeros16 = jnp.zeros((16,), jnp.float32)
    ones16 = jnp.ones((16,), jnp.float32)
    lane = lax.iota(jnp.int32, 16)
    lane_pair = lax.shift_right_logical(lane, 3)
    lane_col = lane & 7
    has_extra = s < NEXTRA

    pltpu.sync_copy(ei_hbm.at[0, pl.ds(s * NIT2, NIT2)],
                    src2_v.at[pl.ds(0, NIT2)])
    pltpu.sync_copy(ei_hbm.at[1, pl.ds(s * NIT2, NIT2)],
                    dst2_v.at[pl.ds(0, NIT2)])

    @pl.when(has_extra)
    def _():
        pltpu.sync_copy(ei_hbm.at[0, pl.ds(NIT2 * NS + s, 1)],
                        src2_v.at[pl.ds(NIT2, 1)])
        pltpu.sync_copy(ei_hbm.at[1, pl.ds(NIT2 * NS + s, 1)],
                        dst2_v.at[pl.ds(NIT2, 1)])

    pltpu.sync_copy(b1_hbm.at[c], b1_v)

    @pl.when(s < NS - 1)
    def _():
        pltpu.sync_copy(batch_hbm.at[pl.ds(row0, RPT)], batch_v)

    @pl.when(s == NS - 1)
    def _():
        nlast = N - (NS - 1) * RPT
        pltpu.sync_copy(batch_hbm.at[pl.ds(row0, nlast)],
                        batch_v.at[pl.ds(0, nlast)])

    def zero_hist(i, carry):
        hist_v[pl.ds(i * 16, 16)] = zeros16
        return carry

    lax.fori_loop(0, NP // 16, zero_hist, 0)

    def deg_body(i, carry):
        for k in range(CHUNK // 16):
            iv = dst2_v[i, pl.ds(k * 16, 16)]
            plsc.addupdate_scatter(hist_v, [iv], ones16)
        return carry

    lax.fori_loop(0, NIT2, deg_body, 0)

    @pl.when(has_extra)
    def _():
        for k in range(CHUNK // 16):
            iv = dst2_v[NIT2, pl.ds(k * 16, 16)]
            plsc.addupdate_scatter(hist_v, [iv], ones16)

    pltpu.sync_copy(hist_v, hist_sh.at[s])
    plsc.subcore_barrier()

    for t in range(NS):
        pltpu.sync_copy(hist_sh.at[t, pl.ds(row0, RPT)],
                        hred_v.at[t, pl.ds(0, RPT)])

    magic = jnp.full((16,), 0x5F3759DF, jnp.int32)

    def dis_body(j, carry):
        d = hred_v[0, pl.ds(j * 16, 16)]
        for t in range(1, NS):
            d = d + hred_v[t, pl.ds(j * 16, 16)]
        d = d + 1.0
        bits = plsc.bitcast(d, jnp.int32)
        y = plsc.bitcast(magic - lax.shift_right_logical(bits, 1),
                         jnp.float32)
        for _ in range(3):
            y = y * (1.5 - 0.5 * d * y * y)
        dis_v[pl.ds(j * 16, 16)] = y
        return carry

    lax.fori_loop(0, RPT // 16, dis_body, 0)

    pltpu.sync_copy(h_hbm.at[c, :, pl.ds(row0, RPT)], hsT_v)
    jconsts = [jnp.full((16,), j, jnp.int32) for j in range(COLS)]

    def l1_body(g, carry):
        dvv = dis_v[pl.ds(g * 16, 16)]
        idxr = lane + g * 16
        for j in range(COLS):
            v = hsT_v[j, pl.ds(g * 16, 16)]
            plsc.store_scatter(ys_v, [idxr, jconsts[j]], v * dvv)
        return carry

    lax.fori_loop(0, RPT // 16, l1_body, 0)
    pltpu.sync_copy(ys_v, table_sh.at[pl.ds(row0, RPT)])
    pltpu.sync_copy(ys_v, acc_sh.at[pl.ds(row0, RPT)])
    plsc.subcore_barrier()

    sem_g = (sem_g0, sem_g1)
    sem_s = (sem_s0, sem_s1)

    def fire_gathers(blk, p):
        for kk in range(KB):
            ch = blk * KB + kk
            pltpu.async_copy(table_sh.at[src2_v.at[ch]], msg_v.at[p, kk],
                             sem_g[p])

    def drain_gathers(p):
        for kk in range(KB):
            pltpu.make_async_copy(table_sh.at[pl.ds(0, CHUNK)],
                                  msg_v.at[p, kk], sem_g[p]).wait()

    def fire_scatters(blk, p):
        for kk in range(KB):
            ch = blk * KB + kk
            pltpu.async_copy(msg_v.at[p, kk], acc_sh.at[dst2_v.at[ch]],
                             sem_s[p], add=True)

    def drain_scatters(p):
        for kk in range(KB):
            pltpu.make_async_copy(table_sh.at[pl.ds(0, CHUNK)],
                                  msg_v.at[p, kk], sem_s[p]).wait()

    def edge_loop():
        fire_gathers(0, 0)

        def pair_body(i, carry):
            blk0 = i * 2

            @pl.when(i > 0)
            def _():
                drain_scatters(1)

            fire_gathers(blk0 + 1, 1)
            drain_gathers(0)
            fire_scatters(blk0, 0)

            drain_scatters(0)
            fire_gathers(blk0 + 2, 0)
            drain_gathers(1)
            fire_scatters(blk0 + 1, 1)
            return carry

        lax.fori_loop(0, NPAIR, pair_body, 0)

        drain_scatters(1)
        drain_gathers(0)
        fire_scatters(NBLK - 1, 0)
        drain_scatters(0)

        @pl.when(has_extra)
        def _():
            pltpu.async_copy(table_sh.at[src2_v.at[NIT2]], msg_v.at[0, 0],
                             sem_g0).wait()
            pltpu.async_copy(msg_v.at[0, 0], acc_sh.at[dst2_v.at[NIT2]],
                             sem_s0, add=True).wait()

    edge_loop()
    plsc.subcore_barrier()

    pltpu.sync_copy(acc_sh.at[pl.ds(row0, RPT)], hs_v)
    b1v = b1_v[...]

    def relu_body(j, carry):
        idxr = lane_pair + j * 2
        dv = plsc.load_gather(dis_v, [idxr])
        av = plsc.load_gather(hs_v, [idxr, lane_col])
        r = jnp.maximum(av * dv + b1v, 0.0)
        plsc.store_scatter(ys_v, [idxr, lane_col], r * dv)
        return carry

    lax.fori_loop(0, RPT // 2, relu_body, 0)
    pltpu.sync_copy(ys_v, table_sh.at[pl.ds(row0, RPT)])
    pltpu.sync_copy(ys_v, acc_sh.at[pl.ds(row0, RPT)])
    plsc.subcore_barrier()

    edge_loop()
    plsc.subcore_barrier()

    def zero_pool(i, carry):
        pool_v[pl.ds(i * 16, 16)] = zeros16
        return carry

    lax.fori_loop(0, 2 * G * COLS // 16, zero_pool, 0)

    pltpu.sync_copy(acc_sh.at[pl.ds(row0, RPT)], hs_v)

    def pool_body(j, carry):
        idxr = lane_pair + j * 2
        dv = plsc.load_gather(dis_v, [idxr])
        av = plsc.load_gather(hs_v, [idxr, lane_col])
        seg = plsc.load_gather(batch_v, [idxr])
        idx = seg * COLS + lane_col
        plsc.addupdate_scatter(pool_v, [idx], av * dv)
        plsc.addupdate_scatter(pool_v, [idx + G * COLS], ones16)
        return carry

    npairs = jnp.where(s == NS - 1, (N - (NS - 1) * RPT) // 2, RPT // 2)
    lax.fori_loop(0, npairs, pool_body, 0)

    pltpu.sync_copy(pool_v, hist_sh.at[s, pl.ds(0, 2 * G * COLS)])
    plsc.subcore_barrier()

    @pl.when(s == 0)
    def _():
        for t in range(NS):
            pltpu.sync_copy(hist_sh.at[t, pl.ds(0, 2 * G * COLS)],
                            hred_v.at[t, pl.ds(0, 2 * G * COLS)])

        def red_body(j, carry):
            v = hred_v[0, pl.ds(j * 16, 16)]
            for t in range(1, NS):
                v = v + hred_v[t, pl.ds(j * 16, 16)]
            pool_v[pl.ds(j * 16, 16)] = v
            return carry

        lax.fori_loop(0, 2 * G * COLS // 16, red_body, 0)
        pltpu.sync_copy(pool_v, out_hbm.at[c])


def _tc_a_body(x_ref, w1_ref, h_ref):
    ht = lax.dot_general(w1_ref[...], x_ref[...], (((0,), (1,)), ((), ())),
                         preferred_element_type=jnp.float32)
    h_ref[0, :, :N] = ht[0:COLS]
    h_ref[1, :, :N] = ht[COLS:H]
    h_ref[0, :, N:NP] = jnp.zeros((COLS, NP - N), jnp.float32)
    h_ref[1, :, N:NP] = jnp.zeros((COLS, NP - N), jnp.float32)


def _tc_b_body(sums_ref, counts_ref, w2_ref, b2_ref, out_ref):
    sums16 = jnp.concatenate([sums_ref[0], sums_ref[1]], axis=1)
    z = jnp.dot(sums16, w2_ref[...], preferred_element_type=jnp.float32)
    pooled = z / jnp.maximum(counts_ref[...], 1.0) + b2_ref[...]
    mx = jnp.max(pooled, axis=1, keepdims=True)
    lse = mx + jnp.log(jnp.sum(jnp.exp(pooled - mx), axis=1, keepdims=True))
    out_ref[...] = pooled - lse


_tc_a = pl.pallas_call(
    _tc_a_body,
    out_shape=jax.ShapeDtypeStruct((NC, COLS, NP), jnp.float32),
)

_tc_b = pl.pallas_call(
    _tc_b_body,
    out_shape=jax.ShapeDtypeStruct((G, C), jnp.float32),
)


@functools.cache
def _sc_kernels():
    mesh = plsc.VectorSubcoreMesh(core_axis_name="c", subcore_axis_name="s",
                                  num_cores=NC, num_subcores=NS)
    sc_fused = pl.kernel(
        _sc_fused_body,
        out_type=jax.ShapeDtypeStruct((NC, 2 * G * COLS), jnp.float32),
        mesh=mesh,
        scratch_types=[
            pltpu.VMEM((NIT2 + 1, CHUNK), jnp.int32),
            pltpu.VMEM((NIT2 + 1, CHUNK), jnp.int32),
            pltpu.VMEM((2, KB, CHUNK, COLS), jnp.float32),
            pltpu.VMEM((NP,), jnp.float32),
            pltpu.VMEM((NS, 2 * G * COLS), jnp.float32),
            pltpu.VMEM((RPT,), jnp.float32),
            pltpu.VMEM((RPT, COLS), jnp.float32),
            pltpu.VMEM((RPT, COLS), jnp.float32),
            pltpu.VMEM((COLS, RPT), jnp.float32),
            pltpu.VMEM((RPT,), jnp.int32),
            pltpu.VMEM((2 * G * COLS,), jnp.float32),
            pltpu.VMEM((16,), jnp.float32),
            pltpu.VMEM_SHARED((NS, NP), jnp.float32),
            pltpu.VMEM_SHARED((NP, COLS), jnp.float32),
            pltpu.VMEM_SHARED((NP, COLS), jnp.float32),
            pltpu.SemaphoreType.DMA,
            pltpu.SemaphoreType.DMA,
            pltpu.SemaphoreType.DMA,
            pltpu.SemaphoreType.DMA,
        ],
        compiler_params=pltpu.CompilerParams(needs_layout_passes=False,
                                             use_tc_tiling_on_sc=False),
    )
    return sc_fused


@jax.jit
def kernel(x, edge_index, batch, W1, b1, W2, b2):
    sc_fused = _sc_kernels()
    ei3 = edge_index.reshape(2, NROWS, CHUNK)
    h = _tc_a(x, W1)
    b1pat = jnp.stack([jnp.tile(b1[0:COLS], 2), jnp.tile(b1[COLS:H], 2)])
    pooled = sc_fused(h, ei3, b1pat, batch)
    sums = pooled[:, :G * COLS].reshape(NC, G, COLS)
    counts = pooled[0, G * COLS:].reshape(G, COLS)[:, 0:1]
    return _tc_b(sums, counts, W2, b2.reshape(1, C))
